# baseline (speedup 1.0000x reference)
import functools

import jax
import jax.numpy as jnp
from jax import lax
from jax.experimental import pallas as pl
from jax.experimental.pallas import tpu as pltpu
from jax.experimental.pallas import tpu_sc as plsc

EMBED_DIM = 64
NUM_CORES = 2
NUM_SUBCORES = 16
NUM_WORKERS = NUM_CORES * NUM_SUBCORES
CHUNK = 256
LANES = 16


def _make_gather(total_rows: int):
    rows_per_worker = total_rows // NUM_WORKERS
    n_chunks = rows_per_worker // CHUNK
    mesh = plsc.VectorSubcoreMesh(core_axis_name="c", subcore_axis_name="s")

    @functools.partial(
        pl.kernel,
        mesh=mesh,
        compiler_params=pltpu.CompilerParams(needs_layout_passes=False),
        out_type=jax.ShapeDtypeStruct((total_rows, EMBED_DIM), jnp.float32),
        scratch_types=[
            pltpu.VMEM((CHUNK,), jnp.int32),
            pltpu.VMEM((CHUNK,), jnp.int32),
            pltpu.VMEM((CHUNK, 2 * EMBED_DIM), jnp.float32),
            pltpu.VMEM((CHUNK, EMBED_DIM), jnp.float32),
            pltpu.SemaphoreType.DMA,
        ],
    )
    def gather(table2_hbm, idx_hbm, out_hbm, idx_v, q_v, g_v, h_v, sem):
        wid = lax.axis_index("s") * NUM_CORES + lax.axis_index("c")
        base = wid * rows_per_worker

        def chunk_body(i, carry):
            off = base + i * CHUNK
            pltpu.sync_copy(idx_hbm.at[pl.ds(off, CHUNK)], idx_v)

            def halve_body(t, carry2):
                sl = pl.ds(t * LANES, LANES)
                q_v[sl] = lax.shift_right_logical(idx_v[sl], 1)
                return carry2

            lax.fori_loop(0, CHUNK // LANES, halve_body, 0)
            pltpu.async_copy(table2_hbm.at[q_v], g_v, sem).wait()

            def sel_body(t, carry2):
                sl = pl.ds(t * LANES, LANES)
                idx16 = idx_v[sl]
                rows16 = lax.iota(jnp.int32, LANES) + t * LANES
                colbase = lax.shift_left(
                    lax.bitwise_and(idx16, jnp.int32(1)), 6
                )
                for j in range(EMBED_DIM):
                    v = plsc.load_gather(g_v, [rows16, colbase + j])
                    plsc.store_scatter(
                        h_v, [rows16, jnp.full((LANES,), j, jnp.int32)], v
                    )
                return carry2

            lax.fori_loop(0, CHUNK // LANES, sel_body, 0)

            pltpu.sync_copy(h_v, out_hbm.at[pl.ds(off, CHUNK)])
            return carry

        lax.fori_loop(0, n_chunks, chunk_body, 0)

    return gather


def kernel(x, table):
    idx = x.reshape(-1).astype(jnp.int32)
    total = idx.shape[0]
    table2 = table.reshape(table.shape[0] // 2, 2 * EMBED_DIM)
    out = _make_gather(total)(table2, idx)
    return out.reshape(x.shape + (EMBED_DIM,))


# speedup vs baseline: 2.1787x; 2.1787x over previous
import functools

import jax
import jax.numpy as jnp
from jax import lax
from jax.experimental import pallas as pl
from jax.experimental.pallas import tpu as pltpu
from jax.experimental.pallas import tpu_sc as plsc

EMBED_DIM = 64
NUM_CORES = 2
NUM_SUBCORES = 16
NUM_WORKERS = NUM_CORES * NUM_SUBCORES
CHUNK = 256
LANES = 16

BATCH = 4096
SEQ = 200


def _make_gather(vocab: int):
    total_rows = BATCH * SEQ
    rows_per_worker = total_rows // NUM_WORKERS
    n_chunks = rows_per_worker // CHUNK
    mesh = plsc.VectorSubcoreMesh(core_axis_name="c", subcore_axis_name="s")

    @functools.partial(
        pl.kernel,
        mesh=mesh,
        compiler_params=pltpu.CompilerParams(
            use_tc_tiling_on_sc=False, needs_layout_passes=False
        ),
        out_type=jax.ShapeDtypeStruct(
            (SEQ, 8, BATCH // 128, 8, 128), jnp.float32
        ),
        scratch_types=[
            pltpu.VMEM((CHUNK,), jnp.int32),
            pltpu.VMEM((CHUNK, EMBED_DIM), jnp.float32),
            pltpu.VMEM((8, 8, CHUNK), jnp.float32),
            pltpu.SemaphoreType.DMA,
        ],
    )
    def gather(table_hbm, idxt_hbm, out_hbm, idx_v, rows_v, t_v, sem):
        wid = lax.axis_index("s") * NUM_CORES + lax.axis_index("c")
        base = wid * rows_per_worker

        def chunk_body(i, carry):
            off = base + i * CHUNK
            s_pos = lax.shift_right_logical(off, 12)
            k0 = lax.shift_right_logical(
                lax.bitwise_and(off, jnp.int32(BATCH - 1)), 7
            )
            pltpu.sync_copy(idxt_hbm.at[pl.ds(off, CHUNK)], idx_v)
            pltpu.async_copy(table_hbm.at[idx_v], rows_v, sem).wait()

            def tr_body(t, carry2):
                rows16 = lax.iota(jnp.int32, LANES) + t * LANES
                for m in range(EMBED_DIM // LANES):
                    for d in range(LANES):
                        cvec = m * LANES + lax.bitwise_and(
                            lax.iota(jnp.int32, LANES) + d, jnp.int32(LANES - 1)
                        )
                        v = plsc.load_gather(rows_v, [rows16, cvec])
                        plsc.store_scatter(
                            t_v,
                            [
                                lax.shift_right_logical(cvec, 3),
                                lax.bitwise_and(cvec, jnp.int32(7)),
                                rows16,
                            ],
                            v,
                        )
                return carry2

            lax.fori_loop(0, CHUNK // LANES, tr_body, 0)

            for kk in range(CHUNK // 128):
                pltpu.sync_copy(
                    t_v.at[:, :, pl.ds(kk * 128, 128)],
                    out_hbm.at[s_pos, :, k0 + kk, :, :],
                )
            return carry

        lax.fori_loop(0, n_chunks, chunk_body, 0)

    return gather


def kernel(x, table):
    idx_t = jnp.swapaxes(x, 0, 1).reshape(-1).astype(jnp.int32)
    vocab = table.shape[0]
    out5 = _make_gather(vocab)(table, idx_t)
    return out5.transpose((2, 4, 0, 1, 3)).reshape(BATCH, SEQ, EMBED_DIM)


# speedup vs baseline: 2.8162x; 1.2926x over previous
import functools

import jax
import jax.numpy as jnp
from jax import lax
from jax.experimental import pallas as pl
from jax.experimental.pallas import tpu as pltpu
from jax.experimental.pallas import tpu_sc as plsc

EMBED_DIM = 64
NUM_CORES = 2
NUM_SUBCORES = 16
NUM_WORKERS = NUM_CORES * NUM_SUBCORES
CHUNK = 256
TPAD = CHUNK + 1
LANES = 16

BATCH = 4096
SEQ = 200


def _make_gather(vocab: int):
    total_rows = BATCH * SEQ
    rows_per_worker = total_rows // NUM_WORKERS
    n_chunks = rows_per_worker // CHUNK
    mesh = plsc.VectorSubcoreMesh(core_axis_name="c", subcore_axis_name="s")

    @functools.partial(
        pl.kernel,
        mesh=mesh,
        compiler_params=pltpu.CompilerParams(
            use_tc_tiling_on_sc=False, needs_layout_passes=False
        ),
        out_type=jax.ShapeDtypeStruct(
            (SEQ, 8, BATCH // 128, 8, 128), jnp.float32
        ),
        scratch_types=[
            pltpu.VMEM((2, CHUNK), jnp.int32),
            pltpu.VMEM((2, CHUNK, EMBED_DIM), jnp.float32),
            pltpu.VMEM((2, 8, 8, TPAD), jnp.float32),
            pltpu.SemaphoreType.DMA,
            pltpu.SemaphoreType.DMA,
            pltpu.SemaphoreType.DMA,
            pltpu.SemaphoreType.DMA,
        ],
    )
    def gather(
        table_hbm, idxt_hbm, out_hbm, idx_v, rows_v, t_v, g0, g1, w0, w1
    ):
        wid = lax.axis_index("s") * NUM_CORES + lax.axis_index("c")
        base = wid * rows_per_worker
        gsems = (g0, g1)
        wsems = (w0, w1)

        lane = lax.iota(jnp.int32, LANES)
        uvecs = []
        c8vecs = []
        for m in range(EMBED_DIM // LANES):
            c = lane + m * LANES
            uvecs.append(lax.shift_right_logical(c, 3))
            c8vecs.append(lax.bitwise_and(c, jnp.int32(7)))

        def start_gather(i, b):
            off = base + i * CHUNK
            pltpu.sync_copy(idxt_hbm.at[pl.ds(off, CHUNK)], idx_v.at[b])
            pltpu.async_copy(table_hbm.at[idx_v.at[b]], rows_v.at[b], gsems[b])

        def out_slices(i, b):
            off = base + i * CHUNK
            s_pos = lax.shift_right_logical(off, 12)
            k0 = lax.shift_right_logical(
                lax.bitwise_and(off, jnp.int32(BATCH - 1)), 7
            )
            pairs = []
            for kk in range(CHUNK // 128):
                pairs.append(
                    (
                        t_v.at[b, :, :, pl.ds(kk * 128, 128)],
                        out_hbm.at[s_pos, :, k0 + kk, :, :],
                    )
                )
            return pairs

        def transpose_chunk(b):
            rows_b = rows_v.at[b]
            t_b = t_v.at[b]

            def tr_body(j, carry):
                for jj in range(2):
                    row = j * 2 + jj
                    bvec = jnp.full((LANES,), 0, jnp.int32) + row
                    for m in range(EMBED_DIM // LANES):
                        v = rows_b[row, pl.ds(m * LANES, LANES)]
                        plsc.store_scatter(
                            t_b, [uvecs[m], c8vecs[m], bvec], v
                        )
                return carry

            lax.fori_loop(0, CHUNK // 2, tr_body, 0)

        start_gather(0, 0)

        def pipe_body(g, carry):
            for b in range(2):
                i = g * 2 + b
                nxt = jnp.minimum(i + 1, n_chunks - 1)

                @pl.when(i + 1 <= n_chunks - 1 if b == 0 else g < n_chunks // 2 - 1)
                def _():
                    start_gather(nxt, 1 - b)

                pltpu.make_async_copy(
                    table_hbm.at[idx_v.at[b]], rows_v.at[b], gsems[b]
                ).wait()

                @pl.when(g >= 1)
                def _():
                    for src, dst in out_slices(i, b):
                        pltpu.make_async_copy(src, dst, wsems[b]).wait()

                transpose_chunk(b)
                for src, dst in out_slices(i, b):
                    pltpu.async_copy(src, dst, wsems[b])
            return carry

        lax.fori_loop(0, n_chunks // 2, pipe_body, 0)
        for b in range(2):
            for src, dst in out_slices(n_chunks - 2 + b, b):
                pltpu.make_async_copy(src, dst, wsems[b]).wait()

    return gather


def kernel(x, table):
    idx_t = jnp.swapaxes(x, 0, 1).reshape(-1).astype(jnp.int32)
    vocab = table.shape[0]
    out5 = _make_gather(vocab)(table, idx_t)
    return out5.transpose((2, 4, 0, 1, 3)).reshape(BATCH, SEQ, EMBED_DIM)


# speedup vs baseline: 2.9181x; 1.0362x over previous
import functools

import jax
import jax.numpy as jnp
from jax import lax
from jax.experimental import pallas as pl
from jax.experimental.pallas import tpu as pltpu
from jax.experimental.pallas import tpu_sc as plsc

EMBED_DIM = 64
NUM_CORES = 2
NUM_SUBCORES = 16
NUM_WORKERS = NUM_CORES * NUM_SUBCORES
CHUNK = 256
TPAD = CHUNK + 1
LANES = 16

BATCH = 4096
SEQ = 200


def _make_gather(vocab: int):
    total_rows = BATCH * SEQ
    rows_per_worker = total_rows // NUM_WORKERS
    n_chunks = rows_per_worker // CHUNK
    mesh = plsc.VectorSubcoreMesh(core_axis_name="c", subcore_axis_name="s")

    @functools.partial(
        pl.kernel,
        mesh=mesh,
        compiler_params=pltpu.CompilerParams(
            use_tc_tiling_on_sc=False, needs_layout_passes=False
        ),
        out_type=jax.ShapeDtypeStruct(
            (SEQ, 8, BATCH // 128, 8, 128), jnp.float32
        ),
        scratch_types=[
            pltpu.VMEM((2, CHUNK), jnp.int32),
            pltpu.VMEM((2, CHUNK, EMBED_DIM), jnp.float32),
            pltpu.VMEM((2, 8, 8, TPAD), jnp.float32),
            pltpu.SemaphoreType.DMA,
            pltpu.SemaphoreType.DMA,
            pltpu.SemaphoreType.DMA,
            pltpu.SemaphoreType.DMA,
        ],
    )
    def gather(
        table_hbm, idxt_hbm, out_hbm, idx_v, rows_v, t_v, g0, g1, w0, w1
    ):
        wid = lax.axis_index("s") * NUM_CORES + lax.axis_index("c")
        base = wid * rows_per_worker
        gsems = (g0, g1)
        wsems = (w0, w1)

        lane = lax.iota(jnp.int32, LANES)
        uvecs = []
        c8vecs = []
        for m in range(EMBED_DIM // LANES):
            c = lane + m * LANES
            uvecs.append(lax.shift_right_logical(c, 3))
            c8vecs.append(lax.bitwise_and(c, jnp.int32(7)))

        def start_gather(i, b):
            off = base + i * CHUNK
            pltpu.sync_copy(idxt_hbm.at[pl.ds(off, CHUNK)], idx_v.at[b])
            pltpu.async_copy(table_hbm.at[idx_v.at[b]], rows_v.at[b], gsems[b])

        def out_slices(i, b):
            off = base + i * CHUNK
            s_pos = lax.shift_right_logical(off, 12)
            k0 = lax.shift_right_logical(
                lax.bitwise_and(off, jnp.int32(BATCH - 1)), 7
            )
            pairs = []
            for kk in range(CHUNK // 128):
                pairs.append(
                    (
                        t_v.at[b, :, :, pl.ds(kk * 128, 128)],
                        out_hbm.at[s_pos, :, k0 + kk, :, :],
                    )
                )
            return pairs

        def transpose_chunk(b):
            rows_b = rows_v.at[b]
            t_b = t_v.at[b]

            def tr_body(j, carry):
                for jj in range(2):
                    row = j * 2 + jj
                    bvec = jnp.full((LANES,), 0, jnp.int32) + row
                    for m in range(EMBED_DIM // LANES):
                        v = rows_b[row, pl.ds(m * LANES, LANES)]
                        plsc.store_scatter(
                            t_b, [uvecs[m], c8vecs[m], bvec], v
                        )
                return carry

            lax.fori_loop(0, CHUNK // 2, tr_body, 0)

        start_gather(0, 0)

        def pipe_body(g, carry):
            for b in range(2):
                i = g * 2 + b
                nxt = jnp.minimum(i + 1, n_chunks - 1)

                @pl.when(i + 1 <= n_chunks - 1 if b == 0 else g < n_chunks // 2 - 1)
                def _():
                    start_gather(nxt, 1 - b)

                pltpu.make_async_copy(
                    table_hbm.at[idx_v.at[b]], rows_v.at[b], gsems[b]
                ).wait()

                @pl.when(g >= 1)
                def _():
                    for src, dst in out_slices(i, b):
                        pltpu.make_async_copy(src, dst, wsems[b]).wait()

                transpose_chunk(b)
                for src, dst in out_slices(i, b):
                    pltpu.async_copy(src, dst, wsems[b])
            return carry

        lax.fori_loop(0, n_chunks // 2, pipe_body, 0)
        for b in range(2):
            for src, dst in out_slices(n_chunks - 2 + b, b):
                pltpu.make_async_copy(src, dst, wsems[b]).wait()

    return gather


TBLK = 4096


def _relayout_table(table):
    vocab = table.shape[0]
    n_blocks = -(-vocab // TBLK)
    tail = vocab - (n_blocks - 1) * TBLK
    t_tr = jnp.swapaxes(table, 0, 1)

    def body(in_ref, eye_ref, out_ref):
        x = in_ref[...]
        tmp = jax.lax.dot_general(
            x,
            eye_ref[...],
            (((0,), (0,)), ((), ())),
            precision=jax.lax.Precision.HIGHEST,
            preferred_element_type=jnp.float32,
        )
        i = pl.program_id(0)

        @pl.when(i < n_blocks - 1)
        def _():
            out_ref[...] = jnp.concatenate(
                [tmp[: TBLK // 2], tmp[TBLK // 2 :]], axis=1
            )

        @pl.when(i == n_blocks - 1)
        def _():
            out_ref[: tail // 2] = jnp.concatenate(
                [tmp[: tail // 2], tmp[tail // 2 : tail]], axis=1
            )

    return pl.pallas_call(
        body,
        grid=(n_blocks,),
        in_specs=[
            pl.BlockSpec((EMBED_DIM, TBLK), lambda i: (0, i)),
            pl.BlockSpec((EMBED_DIM, EMBED_DIM), lambda i: (0, 0)),
        ],
        out_specs=pl.BlockSpec((TBLK // 2, 2 * EMBED_DIM), lambda i: (i, 0)),
        out_shape=jax.ShapeDtypeStruct(
            (vocab // 2, 2 * EMBED_DIM), jnp.float32
        ),
    )(t_tr, jnp.eye(EMBED_DIM, dtype=jnp.float32))


def kernel(x, table):
    idx_t = jnp.swapaxes(x, 0, 1).reshape(-1).astype(jnp.int32)
    vocab = table.shape[0]
    tail_base = (vocab // TBLK) * TBLK
    tail_half = (vocab - tail_base) // 2
    blk = idx_t // TBLK
    a = idx_t - blk * TBLK
    sel_main = blk * TBLK + (a % (TBLK // 2)) * 2 + a // (TBLK // 2)
    at = idx_t - tail_base
    sel_tail = tail_base + (at % tail_half) * 2 + at // tail_half
    sel = jnp.where(idx_t < tail_base, sel_main, sel_tail)
    t2 = _relayout_table(table).reshape(vocab, EMBED_DIM)
    out5 = _make_gather(vocab)(t2, sel)
    return out5.transpose((2, 4, 0, 1, 3)).reshape(BATCH, SEQ, EMBED_DIM)


# speedup vs baseline: 3.7031x; 1.2690x over previous
import functools

import jax
import jax.numpy as jnp
from jax import lax
from jax.experimental import pallas as pl
from jax.experimental.pallas import tpu as pltpu
from jax.experimental.pallas import tpu_sc as plsc

EMBED_DIM = 64
NUM_CORES = 2
NUM_SUBCORES = 16
NUM_WORKERS = NUM_CORES * NUM_SUBCORES
CHUNK = 256
TPAD = CHUNK + 1
LANES = 16

BATCH = 4096
SEQ = 200


def _make_gather(vocab: int):
    total_rows = BATCH * SEQ
    rows_per_worker = total_rows // NUM_WORKERS
    n_chunks = rows_per_worker // CHUNK
    mesh = plsc.VectorSubcoreMesh(core_axis_name="c", subcore_axis_name="s")

    @functools.partial(
        pl.kernel,
        mesh=mesh,
        compiler_params=pltpu.CompilerParams(
            use_tc_tiling_on_sc=False, needs_layout_passes=False
        ),
        out_type=jax.ShapeDtypeStruct(
            (SEQ, 8, BATCH // 128, 8, 128), jnp.float32
        ),
        scratch_types=[
            pltpu.VMEM((2, CHUNK), jnp.int32),
            pltpu.VMEM((2, CHUNK, EMBED_DIM), jnp.float32),
            pltpu.VMEM((2, 8, 8, TPAD), jnp.float32),
            pltpu.SemaphoreType.DMA,
            pltpu.SemaphoreType.DMA,
            pltpu.SemaphoreType.DMA,
            pltpu.SemaphoreType.DMA,
        ],
    )
    def gather(
        table_hbm, idxt_hbm, out_hbm, idx_v, rows_v, t_v, g0, g1, w0, w1
    ):
        wid = lax.axis_index("s") * NUM_CORES + lax.axis_index("c")
        base = wid * rows_per_worker
        gsems = (g0, g1)
        wsems = (w0, w1)

        lane = lax.iota(jnp.int32, LANES)
        uvecs = []
        c8vecs = []
        for m in range(EMBED_DIM // LANES):
            c = lane + m * LANES
            uvecs.append(lax.shift_right_logical(c, 3))
            c8vecs.append(lax.bitwise_and(c, jnp.int32(7)))

        def start_gather(i, b):
            off = base + i * CHUNK
            pltpu.sync_copy(idxt_hbm.at[pl.ds(off, CHUNK)], idx_v.at[b])
            pltpu.async_copy(table_hbm.at[idx_v.at[b]], rows_v.at[b], gsems[b])

        def out_slices(i, b):
            off = base + i * CHUNK
            s_pos = lax.shift_right_logical(off, 12)
            k0 = lax.shift_right_logical(
                lax.bitwise_and(off, jnp.int32(BATCH - 1)), 7
            )
            pairs = []
            for kk in range(CHUNK // 128):
                pairs.append(
                    (
                        t_v.at[b, :, :, pl.ds(kk * 128, 128)],
                        out_hbm.at[s_pos, :, k0 + kk, :, :],
                    )
                )
            return pairs

        def transpose_chunk(b):
            rows_b = rows_v.at[b]
            t_b = t_v.at[b]

            def tr_body(j, carry):
                for jj in range(2):
                    row = j * 2 + jj
                    bvec = jnp.full((LANES,), 0, jnp.int32) + row
                    for m in range(EMBED_DIM // LANES):
                        v = rows_b[row, pl.ds(m * LANES, LANES)]
                        plsc.store_scatter(
                            t_b, [uvecs[m], c8vecs[m], bvec], v
                        )
                return carry

            lax.fori_loop(0, CHUNK // 2, tr_body, 0)

        start_gather(0, 0)

        def pipe_body(g, carry):
            for b in range(2):
                i = g * 2 + b
                nxt = jnp.minimum(i + 1, n_chunks - 1)

                @pl.when(i + 1 <= n_chunks - 1 if b == 0 else g < n_chunks // 2 - 1)
                def _():
                    start_gather(nxt, 1 - b)

                pltpu.make_async_copy(
                    table_hbm.at[idx_v.at[b]], rows_v.at[b], gsems[b]
                ).wait()

                @pl.when(g >= 1)
                def _():
                    for src, dst in out_slices(i, b):
                        pltpu.make_async_copy(src, dst, wsems[b]).wait()

                transpose_chunk(b)
                for src, dst in out_slices(i, b):
                    pltpu.async_copy(src, dst, wsems[b])
            return carry

        lax.fori_loop(0, n_chunks // 2, pipe_body, 0)
        for b in range(2):
            for src, dst in out_slices(n_chunks - 2 + b, b):
                pltpu.make_async_copy(src, dst, wsems[b]).wait()

    return gather


TBLK = 4096


def _relayout_table(table):
    vocab = table.shape[0]
    n_blocks = -(-vocab // TBLK)
    tail = vocab - (n_blocks - 1) * TBLK
    t_tr = jnp.swapaxes(table, 0, 1)

    def body(in_ref, out_ref):
        x = in_ref[...]
        tmp = jnp.transpose(x, (1, 0))
        i = pl.program_id(0)

        @pl.when(i < n_blocks - 1)
        def _():
            out_ref[...] = jnp.concatenate(
                [tmp[: TBLK // 2], tmp[TBLK // 2 :]], axis=1
            )

        @pl.when(i == n_blocks - 1)
        def _():
            out_ref[: tail // 2] = jnp.concatenate(
                [tmp[: tail // 2], tmp[tail // 2 : tail]], axis=1
            )

    return pl.pallas_call(
        body,
        grid=(n_blocks,),
        in_specs=[
            pl.BlockSpec((EMBED_DIM, TBLK), lambda i: (0, i)),
        ],
        out_specs=pl.BlockSpec((TBLK // 2, 2 * EMBED_DIM), lambda i: (i, 0)),
        out_shape=jax.ShapeDtypeStruct(
            (vocab // 2, 2 * EMBED_DIM), jnp.float32
        ),
    )(t_tr)


def kernel(x, table):
    idx_t = jnp.swapaxes(x, 0, 1).reshape(-1).astype(jnp.int32)
    vocab = table.shape[0]
    tail_base = (vocab // TBLK) * TBLK
    tail_half = (vocab - tail_base) // 2
    blk = idx_t // TBLK
    a = idx_t - blk * TBLK
    sel_main = blk * TBLK + (a % (TBLK // 2)) * 2 + a // (TBLK // 2)
    at = idx_t - tail_base
    sel_tail = tail_base + (at % tail_half) * 2 + at // tail_half
    sel = jnp.where(idx_t < tail_base, sel_main, sel_tail)
    t2 = _relayout_table(table).reshape(vocab, EMBED_DIM)
    out5 = _make_gather(vocab)(t2, sel)
    return out5.transpose((2, 4, 0, 1, 3)).reshape(BATCH, SEQ, EMBED_DIM)


# speedup vs baseline: 4.0752x; 1.1005x over previous
import functools

import jax
import jax.numpy as jnp
from jax import lax
from jax.experimental import pallas as pl
from jax.experimental.pallas import tpu as pltpu
from jax.experimental.pallas import tpu_sc as plsc

EMBED_DIM = 64
NUM_CORES = 2
NUM_SUBCORES = 16
NUM_WORKERS = NUM_CORES * NUM_SUBCORES
CHUNK = 256
TPAD = CHUNK + 1
LANES = 16

BATCH = 4096
SEQ = 200


def _make_gather(vocab: int):
    total_rows = BATCH * SEQ
    rows_per_worker = total_rows // NUM_WORKERS
    n_chunks = rows_per_worker // CHUNK
    mesh = plsc.VectorSubcoreMesh(core_axis_name="c", subcore_axis_name="s")

    @functools.partial(
        pl.kernel,
        mesh=mesh,
        compiler_params=pltpu.CompilerParams(
            use_tc_tiling_on_sc=False, needs_layout_passes=False
        ),
        out_type=jax.ShapeDtypeStruct(
            (SEQ, 8, BATCH // 128, 8, 128), jnp.float32
        ),
        scratch_types=[
            pltpu.VMEM((2, CHUNK), jnp.int32),
            pltpu.VMEM((2, CHUNK, EMBED_DIM), jnp.float32),
            pltpu.VMEM((2, 8, 8, TPAD), jnp.float32),
            pltpu.SemaphoreType.DMA,
            pltpu.SemaphoreType.DMA,
            pltpu.SemaphoreType.DMA,
            pltpu.SemaphoreType.DMA,
        ],
    )
    def gather(
        table_hbm, idxt_hbm, out_hbm, idx_v, rows_v, t_v, g0, g1, w0, w1
    ):
        wid = lax.axis_index("s") * NUM_CORES + lax.axis_index("c")
        base = wid * rows_per_worker
        gsems = (g0, g1)
        wsems = (w0, w1)

        lane = lax.iota(jnp.int32, LANES)
        uvecs = []
        c8vecs = []
        for m in range(EMBED_DIM // LANES):
            c = lane + m * LANES
            uvecs.append(lax.shift_right_logical(c, 3))
            c8vecs.append(lax.bitwise_and(c, jnp.int32(7)))

        def start_gather(i, b):
            off = base + i * CHUNK
            pltpu.sync_copy(idxt_hbm.at[pl.ds(off, CHUNK)], idx_v.at[b])
            pltpu.async_copy(table_hbm.at[idx_v.at[b]], rows_v.at[b], gsems[b])

        def out_slices(i, b):
            off = base + i * CHUNK
            s_pos = lax.shift_right_logical(off, 12)
            k0 = lax.shift_right_logical(
                lax.bitwise_and(off, jnp.int32(BATCH - 1)), 7
            )
            pairs = []
            for kk in range(CHUNK // 128):
                pairs.append(
                    (
                        t_v.at[b, :, :, pl.ds(kk * 128, 128)],
                        out_hbm.at[s_pos, :, k0 + kk, :, :],
                    )
                )
            return pairs

        def transpose_chunk(b):
            rows_b = rows_v.at[b]
            t_b = t_v.at[b]

            def tr_body(j, carry):
                for jj in range(8):
                    row = j * 8 + jj
                    bvec = jnp.full((LANES,), 0, jnp.int32) + row
                    for m in range(EMBED_DIM // LANES):
                        v = rows_b[row, pl.ds(m * LANES, LANES)]
                        plsc.store_scatter(
                            t_b, [uvecs[m], c8vecs[m], bvec], v
                        )
                return carry

            lax.fori_loop(0, CHUNK // 8, tr_body, 0)

        start_gather(0, 0)

        def pipe_body(g, carry):
            for b in range(2):
                i = g * 2 + b
                nxt = jnp.minimum(i + 1, n_chunks - 1)

                @pl.when(i + 1 <= n_chunks - 1 if b == 0 else g < n_chunks // 2 - 1)
                def _():
                    start_gather(nxt, 1 - b)

                pltpu.make_async_copy(
                    table_hbm.at[idx_v.at[b]], rows_v.at[b], gsems[b]
                ).wait()

                @pl.when(g >= 1)
                def _():
                    for src, dst in out_slices(i, b):
                        pltpu.make_async_copy(src, dst, wsems[b]).wait()

                transpose_chunk(b)
                for src, dst in out_slices(i, b):
                    pltpu.async_copy(src, dst, wsems[b])
            return carry

        lax.fori_loop(0, n_chunks // 2, pipe_body, 0)
        for b in range(2):
            for src, dst in out_slices(n_chunks - 2 + b, b):
                pltpu.make_async_copy(src, dst, wsems[b]).wait()

    return gather


TBLK = 8192


def _relayout_table(table):
    vocab = table.shape[0]
    n_blocks = -(-vocab // TBLK)
    tail = vocab - (n_blocks - 1) * TBLK
    t_tr = jnp.swapaxes(table, 0, 1)

    def body(in_ref, out_ref):
        x = in_ref[...]
        tmp = jnp.transpose(x, (1, 0))
        i = pl.program_id(0)

        @pl.when(i < n_blocks - 1)
        def _():
            out_ref[...] = jnp.concatenate(
                [tmp[: TBLK // 2], tmp[TBLK // 2 :]], axis=1
            )

        @pl.when(i == n_blocks - 1)
        def _():
            out_ref[: tail // 2] = jnp.concatenate(
                [tmp[: tail // 2], tmp[tail // 2 : tail]], axis=1
            )

    return pl.pallas_call(
        body,
        grid=(n_blocks,),
        in_specs=[
            pl.BlockSpec((EMBED_DIM, TBLK), lambda i: (0, i)),
        ],
        out_specs=pl.BlockSpec((TBLK // 2, 2 * EMBED_DIM), lambda i: (i, 0)),
        out_shape=jax.ShapeDtypeStruct(
            (vocab // 2, 2 * EMBED_DIM), jnp.float32
        ),
    )(t_tr)


def kernel(x, table):
    idx_t = jnp.swapaxes(x, 0, 1).reshape(-1).astype(jnp.int32)
    vocab = table.shape[0]
    tail_base = (vocab // TBLK) * TBLK
    tail_half = (vocab - tail_base) // 2
    blk = idx_t // TBLK
    a = idx_t - blk * TBLK
    sel_main = blk * TBLK + (a % (TBLK // 2)) * 2 + a // (TBLK // 2)
    at = idx_t - tail_base
    sel_tail = tail_base + (at % tail_half) * 2 + at // tail_half
    sel = jnp.where(idx_t < tail_base, sel_main, sel_tail)
    t2 = _relayout_table(table).reshape(vocab, EMBED_DIM)
    out5 = _make_gather(vocab)(t2, sel)
    return out5.transpose((2, 4, 0, 1, 3)).reshape(BATCH, SEQ, EMBED_DIM)


# speedup vs baseline: 4.1702x; 1.0233x over previous
import functools

import jax
import jax.numpy as jnp
from jax import lax
from jax.experimental import pallas as pl
from jax.experimental.pallas import tpu as pltpu
from jax.experimental.pallas import tpu_sc as plsc

EMBED_DIM = 64
NUM_CORES = 2
NUM_SUBCORES = 16
NUM_WORKERS = NUM_CORES * NUM_SUBCORES
CHUNK = 512
TPAD = CHUNK + 1
LANES = 16

BATCH = 4096
SEQ = 200


def _make_gather(vocab: int):
    total_rows = BATCH * SEQ
    rows_per_worker = total_rows // NUM_WORKERS
    n_chunks = rows_per_worker // CHUNK
    mesh = plsc.VectorSubcoreMesh(core_axis_name="c", subcore_axis_name="s")

    @functools.partial(
        pl.kernel,
        mesh=mesh,
        compiler_params=pltpu.CompilerParams(
            use_tc_tiling_on_sc=False, needs_layout_passes=False
        ),
        out_type=jax.ShapeDtypeStruct(
            (SEQ, 8, BATCH // 128, 8, 128), jnp.float32
        ),
        scratch_types=[
            pltpu.VMEM((2, CHUNK), jnp.int32),
            pltpu.VMEM((2, CHUNK, EMBED_DIM), jnp.float32),
            pltpu.VMEM((8, 8, TPAD), jnp.float32),
            pltpu.SemaphoreType.DMA,
            pltpu.SemaphoreType.DMA,
            pltpu.SemaphoreType.DMA,
        ],
    )
    def gather(table_hbm, idxt_hbm, out_hbm, idx_v, rows_v, t_v, g0, g1, wsem):
        wid = lax.axis_index("s") * NUM_CORES + lax.axis_index("c")
        base = wid * rows_per_worker
        gsems = (g0, g1)

        lane = lax.iota(jnp.int32, LANES)
        uvecs = []
        c8vecs = []
        for m in range(EMBED_DIM // LANES):
            c = lane + m * LANES
            uvecs.append(lax.shift_right_logical(c, 3))
            c8vecs.append(lax.bitwise_and(c, jnp.int32(7)))

        def start_gather(i, b):
            off = base + i * CHUNK
            pltpu.sync_copy(idxt_hbm.at[pl.ds(off, CHUNK)], idx_v.at[b])
            pltpu.async_copy(table_hbm.at[idx_v.at[b]], rows_v.at[b], gsems[b])

        def out_slices(i):
            off = base + i * CHUNK
            s_pos = lax.shift_right_logical(off, 12)
            k0 = lax.shift_right_logical(
                lax.bitwise_and(off, jnp.int32(BATCH - 1)), 7
            )
            pairs = []
            for kk in range(CHUNK // 128):
                pairs.append(
                    (
                        t_v.at[:, :, pl.ds(kk * 128, 128)],
                        out_hbm.at[s_pos, :, k0 + kk, :, :],
                    )
                )
            return pairs

        def transpose_chunk(b):
            rows_b = rows_v.at[b]

            def tr_body(j, carry):
                for jj in range(8):
                    row = j * 8 + jj
                    bvec = jnp.full((LANES,), 0, jnp.int32) + row
                    for m in range(EMBED_DIM // LANES):
                        v = rows_b[row, pl.ds(m * LANES, LANES)]
                        plsc.store_scatter(
                            t_v, [uvecs[m], c8vecs[m], bvec], v
                        )
                return carry

            lax.fori_loop(0, CHUNK // 8, tr_body, 0)

        start_gather(0, 0)

        def pipe_body(g, carry):
            for b in range(2):
                i = g * 2 + b

                @pl.when(i + 1 <= n_chunks - 1 if b == 0 else g < n_chunks // 2 - 1)
                def _():
                    start_gather(jnp.minimum(i + 1, n_chunks - 1), 1 - b)

                pltpu.make_async_copy(
                    table_hbm.at[idx_v.at[b]], rows_v.at[b], gsems[b]
                ).wait()

                @pl.when(i >= 1)
                def _():
                    for src, dst in out_slices(i):
                        pltpu.make_async_copy(src, dst, wsem).wait()

                transpose_chunk(b)
                for src, dst in out_slices(i):
                    pltpu.async_copy(src, dst, wsem)
            return carry

        lax.fori_loop(0, n_chunks // 2, pipe_body, 0)
        for src, dst in out_slices(n_chunks - 1):
            pltpu.make_async_copy(src, dst, wsem).wait()

    return gather


TBLK = 8192


def _relayout_table(table):
    vocab = table.shape[0]
    n_blocks = -(-vocab // TBLK)
    tail = vocab - (n_blocks - 1) * TBLK
    t_tr = jnp.swapaxes(table, 0, 1)

    def body(in_ref, out_ref):
        x = in_ref[...]
        tmp = jnp.transpose(x, (1, 0))
        i = pl.program_id(0)

        @pl.when(i < n_blocks - 1)
        def _():
            out_ref[...] = jnp.concatenate(
                [tmp[: TBLK // 2], tmp[TBLK // 2 :]], axis=1
            )

        @pl.when(i == n_blocks - 1)
        def _():
            out_ref[: tail // 2] = jnp.concatenate(
                [tmp[: tail // 2], tmp[tail // 2 : tail]], axis=1
            )

    return pl.pallas_call(
        body,
        grid=(n_blocks,),
        in_specs=[
            pl.BlockSpec((EMBED_DIM, TBLK), lambda i: (0, i)),
        ],
        out_specs=pl.BlockSpec((TBLK // 2, 2 * EMBED_DIM), lambda i: (i, 0)),
        out_shape=jax.ShapeDtypeStruct(
            (vocab // 2, 2 * EMBED_DIM), jnp.float32
        ),
    )(t_tr)


def kernel(x, table):
    idx_t = jnp.swapaxes(x, 0, 1).reshape(-1).astype(jnp.int32)
    vocab = table.shape[0]
    tail_base = (vocab // TBLK) * TBLK
    tail_half = (vocab - tail_base) // 2
    blk = idx_t // TBLK
    a = idx_t - blk * TBLK
    sel_main = blk * TBLK + (a % (TBLK // 2)) * 2 + a // (TBLK // 2)
    at = idx_t - tail_base
    sel_tail = tail_base + (at % tail_half) * 2 + at // tail_half
    sel = jnp.where(idx_t < tail_base, sel_main, sel_tail)
    t2 = _relayout_table(table).reshape(vocab, EMBED_DIM)
    out5 = _make_gather(vocab)(t2, sel)
    return out5.transpose((2, 4, 0, 1, 3)).reshape(BATCH, SEQ, EMBED_DIM)


# speedup vs baseline: 4.3746x; 1.0490x over previous
import functools

import jax
import jax.numpy as jnp
from jax import lax
from jax.experimental import pallas as pl
from jax.experimental.pallas import tpu as pltpu
from jax.experimental.pallas import tpu_sc as plsc

EMBED_DIM = 64
NUM_CORES = 2
NUM_SUBCORES = 16
NUM_WORKERS = NUM_CORES * NUM_SUBCORES
CHUNK = 512
TPAD = CHUNK + 1
LANES = 16

BATCH = 4096
SEQ = 200


def _make_gather(vocab: int):
    total_rows = BATCH * SEQ
    rows_per_worker = total_rows // NUM_WORKERS
    n_chunks = rows_per_worker // CHUNK
    mesh = plsc.VectorSubcoreMesh(core_axis_name="c", subcore_axis_name="s")

    @functools.partial(
        pl.kernel,
        mesh=mesh,
        compiler_params=pltpu.CompilerParams(
            use_tc_tiling_on_sc=False, needs_layout_passes=False
        ),
        out_type=jax.ShapeDtypeStruct(
            (SEQ, 8, BATCH // 128, 8, 128), jnp.float32
        ),
        scratch_types=[
            pltpu.VMEM((2, CHUNK), jnp.int32),
            pltpu.VMEM((2, CHUNK, EMBED_DIM), jnp.float32),
            pltpu.VMEM((8, 8, TPAD), jnp.float32),
            pltpu.SemaphoreType.DMA,
            pltpu.SemaphoreType.DMA,
            pltpu.SemaphoreType.DMA,
        ],
    )
    def gather(table_hbm, idxt_hbm, out_hbm, idx_v, rows_v, t_v, g0, g1, wsem):
        wid = lax.axis_index("s") * NUM_CORES + lax.axis_index("c")
        base = wid * rows_per_worker
        gsems = (g0, g1)

        lane = lax.iota(jnp.int32, LANES)
        uvecs = []
        c8vecs = []
        for m in range(EMBED_DIM // LANES):
            c = lane + m * LANES
            uvecs.append(lax.shift_right_logical(c, 3))
            c8vecs.append(lax.bitwise_and(c, jnp.int32(7)))

        def start_gather(i, b):
            off = base + i * CHUNK
            pltpu.sync_copy(idxt_hbm.at[pl.ds(off, CHUNK)], idx_v.at[b])
            pltpu.async_copy(table_hbm.at[idx_v.at[b]], rows_v.at[b], gsems[b])

        def out_slices(i):
            off = base + i * CHUNK
            s_pos = lax.shift_right_logical(off, 12)
            k0 = lax.shift_right_logical(
                lax.bitwise_and(off, jnp.int32(BATCH - 1)), 7
            )
            pairs = []
            for kk in range(CHUNK // 128):
                pairs.append(
                    (
                        t_v.at[:, :, pl.ds(kk * 128, 128)],
                        out_hbm.at[s_pos, :, k0 + kk, :, :],
                    )
                )
            return pairs

        def transpose_chunk(b):
            rows_b = rows_v.at[b]

            def tr_body(j, carry):
                for jj in range(8):
                    row = j * 8 + jj
                    bvec = jnp.full((LANES,), 0, jnp.int32) + row
                    for m in range(EMBED_DIM // LANES):
                        v = rows_b[row, pl.ds(m * LANES, LANES)]
                        plsc.store_scatter(
                            t_v, [uvecs[m], c8vecs[m], bvec], v
                        )
                return carry

            lax.fori_loop(0, CHUNK // 8, tr_body, 0)

        start_gather(0, 0)

        def pipe_body(g, carry):
            for b in range(2):
                i = g * 2 + b

                @pl.when(i + 1 <= n_chunks - 1 if b == 0 else g < n_chunks // 2 - 1)
                def _():
                    start_gather(jnp.minimum(i + 1, n_chunks - 1), 1 - b)

                pltpu.make_async_copy(
                    table_hbm.at[idx_v.at[b]], rows_v.at[b], gsems[b]
                ).wait()

                @pl.when(i >= 1)
                def _():
                    for src, dst in out_slices(i):
                        pltpu.make_async_copy(src, dst, wsem).wait()

                transpose_chunk(b)
                for src, dst in out_slices(i):
                    pltpu.async_copy(src, dst, wsem)
            return carry

        lax.fori_loop(0, n_chunks // 2, pipe_body, 0)
        for src, dst in out_slices(n_chunks - 1):
            pltpu.make_async_copy(src, dst, wsem).wait()

    return gather


TBLK = 16384


def _relayout_table(table):
    vocab = table.shape[0]
    n_blocks = -(-vocab // TBLK)
    tail = vocab - (n_blocks - 1) * TBLK
    t_tr = jnp.swapaxes(table, 0, 1)

    def body(in_ref, out_ref):
        x = in_ref[...]
        tmp = jnp.transpose(x, (1, 0))
        i = pl.program_id(0)

        @pl.when(i < n_blocks - 1)
        def _():
            out_ref[...] = jnp.concatenate(
                [tmp[: TBLK // 2], tmp[TBLK // 2 :]], axis=1
            )

        @pl.when(i == n_blocks - 1)
        def _():
            out_ref[: tail // 2] = jnp.concatenate(
                [tmp[: tail // 2], tmp[tail // 2 : tail]], axis=1
            )

    return pl.pallas_call(
        body,
        grid=(n_blocks,),
        in_specs=[
            pl.BlockSpec((EMBED_DIM, TBLK), lambda i: (0, i)),
        ],
        out_specs=pl.BlockSpec((TBLK // 2, 2 * EMBED_DIM), lambda i: (i, 0)),
        out_shape=jax.ShapeDtypeStruct(
            (vocab // 2, 2 * EMBED_DIM), jnp.float32
        ),
    )(t_tr)


def kernel(x, table):
    idx_t = jnp.swapaxes(x, 0, 1).reshape(-1).astype(jnp.int32)
    vocab = table.shape[0]
    tail_base = (vocab // TBLK) * TBLK
    tail_half = (vocab - tail_base) // 2
    blk = idx_t // TBLK
    a = idx_t - blk * TBLK
    sel_main = blk * TBLK + (a % (TBLK // 2)) * 2 + a // (TBLK // 2)
    at = idx_t - tail_base
    sel_tail = tail_base + (at % tail_half) * 2 + at // tail_half
    sel = jnp.where(idx_t < tail_base, sel_main, sel_tail)
    t2 = _relayout_table(table).reshape(vocab, EMBED_DIM)
    out5 = _make_gather(vocab)(t2, sel)
    return out5.transpose((2, 4, 0, 1, 3)).reshape(BATCH, SEQ, EMBED_DIM)
